# baseline (speedup 1.0000x reference)
import jax
import jax.numpy as jnp
from jax import lax
from jax.experimental import pallas as pl
from jax.experimental.pallas import tpu as pltpu
from jax.experimental.pallas import tpu_sc as plsc

B = 4096
L_HIST = 50
T_TOK = 20
V = 1_000_000
D = 64
M = 100_000

NC, NS, LANES = 2, 16, 16
NW = NC * NS
ROWS_PER_W = B // NW
CHUNK = 8
NCHUNK = ROWS_PER_W // CHUNK
NDIRECT = 49
NSLOT = 50
GROUP = 16
ENT_PER_ROW = NSLOT
ENT_PER_CHUNK = CHUNK * ENT_PER_ROW
ENT_PER_GROUP = GROUP * ENT_PER_ROW

SCALAR_SLOTS = ((0, 0), (1, 1), (2, 0), (3, 0), (4, 2), (5, 3), (6, 4), (7, 5), (8, 6))
TTOK_OFF = 7
QTOK_OFF = 8

IDX_WINDOWS = ((0, 128), (128, 128), (256, 128), (384, 16))


def _gather_body(adid, adv, dep, pos, desc, uid, qid, kid, tid, ttok, qtok,
                 adl_flat, table, out, featb, ttkb, qtkb, adlb, idxs, idxd,
                 obuf0, obuf1, hbuf0, hbuf1,
                 sem_in, sem_g0, sem_g1, sem_h0, sem_h1, sem_o0, sem_o1):
    wid = lax.axis_index("s") * NC + lax.axis_index("c")
    base = wid * ROWS_PER_W
    iota = lax.iota(jnp.int32, LANES)
    obufs = (obuf0, obuf1)
    hbufs = (hbuf0, hbuf1)
    sems_g = (sem_g0, sem_g1)
    sems_h = (sem_h0, sem_h1)
    sems_o = (sem_o0, sem_o1)

    cps = []
    for f, ref in enumerate((adid, adv, dep, pos, desc, uid, qid, kid, tid)):
        cp = pltpu.make_async_copy(
            ref.at[pl.ds(base, ROWS_PER_W)],
            featb.at[pl.ds(f * ROWS_PER_W, ROWS_PER_W)], sem_in)
        cp.start()
        cps.append(cp)
    for src, dst in ((ttok, ttkb), (qtok, qtkb)):
        cp = pltpu.make_async_copy(src.at[pl.ds(base, ROWS_PER_W)], dst, sem_in)
        cp.start()
        cps.append(cp)
    cp = pltpu.make_async_copy(
        adl_flat.at[pl.ds(base * L_HIST, ROWS_PER_W * L_HIST)], adlb, sem_in)
    cp.start()
    cps.append(cp)
    for cp in cps:
        cp.wait()

    idxs[NDIRECT, :] = jnp.zeros((LANES,), jnp.int32)

    def build_group(g, carry):
        rb = g * GROUP
        for s, (fi, off) in enumerate(SCALAR_SLOTS):
            v = featb[pl.ds(fi * ROWS_PER_W + rb, GROUP)]
            idxs[s, :] = v % M + off * M
        for t in range(T_TOK):
            v = plsc.load_gather(ttkb, [rb + iota, jnp.full((LANES,), t, jnp.int32)])
            idxs[9 + t, :] = v % M + TTOK_OFF * M
        for t in range(T_TOK):
            v = plsc.load_gather(qtkb, [rb + iota, jnp.full((LANES,), t, jnp.int32)])
            idxs[29 + t, :] = v % M + QTOK_OFF * M
        for k in range(ENT_PER_GROUP // LANES):
            e = k * LANES + iota
            v = plsc.load_gather(idxs, [e % ENT_PER_ROW, e // ENT_PER_ROW])
            idxd[pl.ds(g * ENT_PER_GROUP + k * LANES, LANES)] = v
        return carry

    lax.fori_loop(0, ROWS_PER_W // GROUP, build_group, 0)

    def fire_g(c):
        p = c % 2
        g_cps = []
        for off, ln in IDX_WINDOWS:
            cp = pltpu.make_async_copy(
                table.at[idxd.at[pl.ds(c * ENT_PER_CHUNK + off, ln)]],
                obufs[p].at[pl.ds(off, ln)], sems_g[p])
            cp.start()
            g_cps.append(cp)
        return g_cps

    def fire_h(c):
        p = c % 2
        h_cps = []
        for off, ln in IDX_WINDOWS:
            cp = pltpu.make_async_copy(
                table.at[adlb.at[pl.ds(c * ENT_PER_CHUNK + off, ln)]],
                hbufs[p].at[pl.ds(off, ln)], sems_h[p])
            cp.start()
            h_cps.append(cp)
        return h_cps

    def pool(c):
        p = c % 2
        hb, ob = hbufs[p], obufs[p]
        for q in range(CHUNK):

            def pool_j(j, acc):
                return tuple(acc[k] + hb[q * L_HIST + j, pl.ds(k * LANES, LANES)]
                             for k in range(D // LANES))

            acc = lax.fori_loop(0, L_HIST, pool_j,
                                tuple(jnp.zeros((LANES,), jnp.float32)
                                      for _ in range(D // LANES)))
            for k in range(D // LANES):
                ob[q * NSLOT + NDIRECT, pl.ds(k * LANES, LANES)] = (
                    acc[k] * (1.0 / L_HIST))

    out_cps = [None, None]
    inflight = [None, None]
    inflight[0] = (fire_g(0), fire_h(0))
    inflight[1] = (fire_g(1), fire_h(1))
    for c in range(NCHUNK):
        p = c % 2
        g_cps, h_cps = inflight[p]
        for cp in g_cps:
            cp.wait()
        for cp in h_cps:
            cp.wait()
        pool(c)
        ocp = pltpu.make_async_copy(
            obufs[p], out.at[pl.ds((base + c * CHUNK) * NSLOT, ENT_PER_CHUNK)],
            sems_o[p])
        ocp.start()
        out_cps[p] = ocp
        if c + 2 < NCHUNK:
            h_next = fire_h(c + 2)
            ocp.wait()
            inflight[p] = (fire_g(c + 2), h_next)
        else:
            ocp.wait()


FMT_CH = 16
FMT_PITCH = 208
XROWS = B * NSLOT // 2


def _fmt_body(x128, out, idxb, obuf, sem_g):
    wid = lax.axis_index("s") * NC + lax.axis_index("c")
    iota = lax.iota(jnp.int32, LANES)

    def build(c, carry):
        blk = wid * FMT_CH + c
        for k in range(FMT_PITCH // LANES):
            n = k * LANES + iota
            v = 200 * blk + 25 * (n & 7) + (n >> 3)
            v = jnp.minimum(v, XROWS - 1)
            idxb[pl.ds(c * FMT_PITCH + k * LANES, LANES)] = v
        return carry

    lax.fori_loop(0, FMT_CH, build, 0)

    def chunk(c, carry):
        blk = wid * FMT_CH + c
        cps = []
        for j in range(25):
            cp = pltpu.make_async_copy(
                x128.at[idxb.at[pl.ds(c * FMT_PITCH + 8 * j, 8)]],
                obuf.at[:, pl.ds(128 * j, 128)], sem_g)
            cp.start()
            cps.append(cp)
        for cp in cps:
            cp.wait()
        pltpu.sync_copy(obuf, out.at[pl.ds(8 * blk, 8)])
        return carry

    lax.fori_loop(0, FMT_CH, chunk, 0)


def kernel(AdID, AdvertiserID, Depth, Position, DescriptionID, user_id,
           QueryID, KeywordID, TitleID, TitleToken, QueryToken, AdIDList, emb_table):
    i32 = jnp.int32
    args = [a.astype(i32) for a in
            (AdID, AdvertiserID, Depth, Position, DescriptionID, user_id,
             QueryID, KeywordID, TitleID, TitleToken, QueryToken)]
    args.append(AdIDList.astype(i32).reshape(-1))
    mesh = plsc.VectorSubcoreMesh(core_axis_name="c", subcore_axis_name="s",
                                  num_cores=NC, num_subcores=NS)
    out = pl.kernel(
        _gather_body,
        out_type=jax.ShapeDtypeStruct((B * NSLOT, D), jnp.float32),
        mesh=mesh,
        compiler_params=pltpu.CompilerParams(needs_layout_passes=False,
                                             use_tc_tiling_on_sc=False,
                                             skip_device_barrier=True),
        scratch_types=[
            pltpu.VMEM((9 * ROWS_PER_W,), i32),
            pltpu.VMEM((ROWS_PER_W, T_TOK), i32),
            pltpu.VMEM((ROWS_PER_W, T_TOK), i32),
            pltpu.VMEM((ROWS_PER_W * L_HIST,), i32),
            pltpu.VMEM((NDIRECT + 1, GROUP), i32),
            pltpu.VMEM((ROWS_PER_W * ENT_PER_ROW,), i32),
            pltpu.VMEM((ENT_PER_CHUNK, D), jnp.float32),
            pltpu.VMEM((ENT_PER_CHUNK, D), jnp.float32),
            pltpu.VMEM((ENT_PER_CHUNK, D), jnp.float32),
            pltpu.VMEM((ENT_PER_CHUNK, D), jnp.float32),
            pltpu.SemaphoreType.DMA,
            pltpu.SemaphoreType.DMA,
            pltpu.SemaphoreType.DMA,
            pltpu.SemaphoreType.DMA,
            pltpu.SemaphoreType.DMA,
            pltpu.SemaphoreType.DMA,
            pltpu.SemaphoreType.DMA,
        ],
    )(*args, emb_table)
    x128 = out.reshape(XROWS, 128)
    z = pl.kernel(
        _fmt_body,
        out_type=jax.ShapeDtypeStruct((B, NSLOT * D), jnp.float32),
        mesh=plsc.VectorSubcoreMesh(core_axis_name="c", subcore_axis_name="s",
                                    num_cores=NC, num_subcores=NS),
        compiler_params=pltpu.CompilerParams(needs_layout_passes=False,
                                             use_tc_tiling_on_sc=True,
                                             skip_device_barrier=True),
        scratch_types=[
            pltpu.VMEM((FMT_CH * FMT_PITCH,), i32),
            pltpu.VMEM((8, NSLOT * D), jnp.float32),
            pltpu.SemaphoreType.DMA,
        ],
    )(x128)
    return z


# speedup vs baseline: 1.0903x; 1.0903x over previous
import jax
import jax.numpy as jnp
from jax import lax
from jax.experimental import pallas as pl
from jax.experimental.pallas import tpu as pltpu
from jax.experimental.pallas import tpu_sc as plsc

B = 4096
L_HIST = 50
T_TOK = 20
V = 1_000_000
D = 64
M = 100_000

NC, NS, LANES = 2, 16, 16
NW = NC * NS
ROWS_PER_W = B // NW
CHUNK = 8
NCHUNK = ROWS_PER_W // CHUNK
NDIRECT = 49
NSLOT = 50
GROUP = 16
ENT_PER_CHUNK = CHUNK * NSLOT
IDX_PITCH = 64

SCALAR_SLOTS = ((0, 0), (1, 1), (2, 0), (3, 0), (4, 2), (5, 3), (6, 4), (7, 5), (8, 6))
TTOK_OFF = 7
QTOK_OFF = 8


def _gather_body(adid, adv, dep, pos, desc, uid, qid, kid, tid, ttok, qtok,
                 adl, table, out, featb, ttkb, qtkb, adlb, idxs, idxd,
                 obuf0, obuf1, hbuf0, hbuf1,
                 sem_in, sem_g0, sem_g1, sem_h0, sem_h1, sem_o0, sem_o1):
    wid = lax.axis_index("s") * NC + lax.axis_index("c")
    base = wid * ROWS_PER_W
    iota = lax.iota(jnp.int32, LANES)
    obufs = (obuf0, obuf1)
    hbufs = (hbuf0, hbuf1)
    sems_g = (sem_g0, sem_g1)
    sems_h = (sem_h0, sem_h1)
    sems_o = (sem_o0, sem_o1)

    cps = []
    for f, ref in enumerate((adid, adv, dep, pos, desc, uid, qid, kid, tid)):
        cp = pltpu.make_async_copy(
            ref.at[pl.ds(base, ROWS_PER_W)],
            featb.at[pl.ds(f * ROWS_PER_W, ROWS_PER_W)], sem_in)
        cp.start()
        cps.append(cp)
    for src, dst in ((ttok, ttkb), (qtok, qtkb)):
        cp = pltpu.make_async_copy(src.at[pl.ds(base, ROWS_PER_W)], dst, sem_in)
        cp.start()
        cps.append(cp)
    cp = pltpu.make_async_copy(adl.at[pl.ds(base, ROWS_PER_W)], adlb, sem_in)
    cp.start()
    cps.append(cp)
    for cp in cps:
        cp.wait()

    for s in range(NDIRECT, IDX_PITCH):
        idxs[s, :] = jnp.zeros((LANES,), jnp.int32)

    def build_group(g, carry):
        rb = g * GROUP
        for s, (fi, off) in enumerate(SCALAR_SLOTS):
            v = featb[pl.ds(fi * ROWS_PER_W + rb, GROUP)]
            idxs[s, :] = v % M + off * M
        for t in range(T_TOK):
            v = plsc.load_gather(ttkb, [rb + iota, jnp.full((LANES,), t, jnp.int32)])
            idxs[9 + t, :] = v % M + TTOK_OFF * M
        for t in range(T_TOK):
            v = plsc.load_gather(qtkb, [rb + iota, jnp.full((LANES,), t, jnp.int32)])
            idxs[29 + t, :] = v % M + QTOK_OFF * M
        for r in range(GROUP):
            for k in range(IDX_PITCH // LANES):
                v = plsc.load_gather(
                    idxs, [k * LANES + iota, jnp.full((LANES,), 1, jnp.int32) * r])
                idxd[pl.ds((rb + r) * IDX_PITCH + k * LANES, LANES)] = v
        return carry

    lax.fori_loop(0, ROWS_PER_W // GROUP, build_group, 0)

    def fire_g(c):
        p = c % 2
        g_cps = []
        for r in range(CHUNK):
            cp = pltpu.make_async_copy(
                table.at[idxd.at[pl.ds((c * CHUNK + r) * IDX_PITCH, NDIRECT)]],
                obufs[p].at[pl.ds(r * NSLOT, NDIRECT)], sems_g[p])
            cp.start()
            g_cps.append(cp)
        return g_cps

    def fire_h(c):
        p = c % 2
        h_cps = []
        for r in range(CHUNK):
            cp = pltpu.make_async_copy(
                table.at[adlb.at[c * CHUNK + r]],
                hbufs[p].at[pl.ds(r * L_HIST, L_HIST)], sems_h[p])
            cp.start()
            h_cps.append(cp)
        return h_cps

    def pool(c):
        p = c % 2
        hb, ob = hbufs[p], obufs[p]
        for q in range(CHUNK):

            def pool_j(j, acc):
                return tuple(acc[k] + hb[q * L_HIST + j, pl.ds(k * LANES, LANES)]
                             for k in range(D // LANES))

            acc = lax.fori_loop(0, L_HIST, pool_j,
                                tuple(jnp.zeros((LANES,), jnp.float32)
                                      for _ in range(D // LANES)))
            for k in range(D // LANES):
                ob[q * NSLOT + NDIRECT, pl.ds(k * LANES, LANES)] = (
                    acc[k] * (1.0 / L_HIST))

    out_cps = [None, None]
    inflight = [None, None]
    inflight[0] = (fire_g(0), fire_h(0))
    inflight[1] = (fire_g(1), fire_h(1))
    for c in range(NCHUNK):
        p = c % 2
        g_cps, h_cps = inflight[p]
        for cp in h_cps:
            cp.wait()
        pool(c)
        for cp in g_cps:
            cp.wait()
        ocp = pltpu.make_async_copy(
            obufs[p], out.at[pl.ds((base + c * CHUNK) * NSLOT, ENT_PER_CHUNK)],
            sems_o[p])
        ocp.start()
        out_cps[p] = ocp
        if c + 2 < NCHUNK:
            h_next = fire_h(c + 2)
            ocp.wait()
            inflight[p] = (fire_g(c + 2), h_next)
        else:
            ocp.wait()


FMT_CH = 16
FMT_PITCH = 208
XROWS = B * NSLOT // 2


def _fmt_body(x128, out, idxb, obuf, sem_g):
    wid = lax.axis_index("s") * NC + lax.axis_index("c")
    iota = lax.iota(jnp.int32, LANES)

    def build(c, carry):
        blk = wid * FMT_CH + c
        for k in range(FMT_PITCH // LANES):
            n = k * LANES + iota
            v = 200 * blk + 25 * (n & 7) + (n >> 3)
            v = jnp.minimum(v, XROWS - 1)
            idxb[pl.ds(c * FMT_PITCH + k * LANES, LANES)] = v
        return carry

    lax.fori_loop(0, FMT_CH, build, 0)

    def chunk(c, carry):
        blk = wid * FMT_CH + c
        cps = []
        for j in range(25):
            cp = pltpu.make_async_copy(
                x128.at[idxb.at[pl.ds(c * FMT_PITCH + 8 * j, 8)]],
                obuf.at[:, pl.ds(128 * j, 128)], sem_g)
            cp.start()
            cps.append(cp)
        for cp in cps:
            cp.wait()
        pltpu.sync_copy(obuf, out.at[pl.ds(8 * blk, 8)])
        return carry

    lax.fori_loop(0, FMT_CH, chunk, 0)


def kernel(AdID, AdvertiserID, Depth, Position, DescriptionID, user_id,
           QueryID, KeywordID, TitleID, TitleToken, QueryToken, AdIDList, emb_table):
    i32 = jnp.int32
    args = [a.astype(i32) for a in
            (AdID, AdvertiserID, Depth, Position, DescriptionID, user_id,
             QueryID, KeywordID, TitleID, TitleToken, QueryToken, AdIDList)]
    mesh = plsc.VectorSubcoreMesh(core_axis_name="c", subcore_axis_name="s",
                                  num_cores=NC, num_subcores=NS)
    out = pl.kernel(
        _gather_body,
        out_type=jax.ShapeDtypeStruct((B * NSLOT, D), jnp.float32),
        mesh=mesh,
        compiler_params=pltpu.CompilerParams(needs_layout_passes=False,
                                             use_tc_tiling_on_sc=False,
                                             skip_device_barrier=True),
        scratch_types=[
            pltpu.VMEM((9 * ROWS_PER_W,), i32),
            pltpu.VMEM((ROWS_PER_W, T_TOK), i32),
            pltpu.VMEM((ROWS_PER_W, T_TOK), i32),
            pltpu.VMEM((ROWS_PER_W, L_HIST), i32),
            pltpu.VMEM((IDX_PITCH, GROUP), i32),
            pltpu.VMEM((ROWS_PER_W * IDX_PITCH,), i32),
            pltpu.VMEM((ENT_PER_CHUNK, D), jnp.float32),
            pltpu.VMEM((ENT_PER_CHUNK, D), jnp.float32),
            pltpu.VMEM((ENT_PER_CHUNK, D), jnp.float32),
            pltpu.VMEM((ENT_PER_CHUNK, D), jnp.float32),
            pltpu.SemaphoreType.DMA,
            pltpu.SemaphoreType.DMA,
            pltpu.SemaphoreType.DMA,
            pltpu.SemaphoreType.DMA,
            pltpu.SemaphoreType.DMA,
            pltpu.SemaphoreType.DMA,
            pltpu.SemaphoreType.DMA,
        ],
    )(*args, emb_table)
    x128 = out.reshape(XROWS, 128)
    z = pl.kernel(
        _fmt_body,
        out_type=jax.ShapeDtypeStruct((B, NSLOT * D), jnp.float32),
        mesh=plsc.VectorSubcoreMesh(core_axis_name="c", subcore_axis_name="s",
                                    num_cores=NC, num_subcores=NS),
        compiler_params=pltpu.CompilerParams(needs_layout_passes=False,
                                             use_tc_tiling_on_sc=True,
                                             skip_device_barrier=True),
        scratch_types=[
            pltpu.VMEM((FMT_CH * FMT_PITCH,), i32),
            pltpu.VMEM((8, NSLOT * D), jnp.float32),
            pltpu.SemaphoreType.DMA,
        ],
    )(x128)
    return z


# speedup vs baseline: 1.0936x; 1.0030x over previous
import jax
import jax.numpy as jnp
from jax import lax
from jax.experimental import pallas as pl
from jax.experimental.pallas import tpu as pltpu
from jax.experimental.pallas import tpu_sc as plsc

B = 4096
L_HIST = 50
T_TOK = 20
V = 1_000_000
D = 64
M = 100_000

NC, NS, LANES = 2, 16, 16
NW = NC * NS
ROWS_PER_W = B // NW
CHUNK = 8
NCHUNK = ROWS_PER_W // CHUNK
NDIRECT = 49
NSLOT = 50
GROUP = 16
ENT_PER_CHUNK = CHUNK * NSLOT
IDX_PITCH = 64

SCALAR_SLOTS = ((0, 0), (1, 1), (2, 0), (3, 0), (4, 2), (5, 3), (6, 4), (7, 5), (8, 6))
TTOK_OFF = 7
QTOK_OFF = 8


def _gather_body(adid, adv, dep, pos, desc, uid, qid, kid, tid, ttok, qtok,
                 adl, table, out, featb, ttkb, qtkb, adlb, idxs, idxd,
                 obuf0, obuf1, hbuf0, hbuf1,
                 sem_in, sem_g0, sem_g1, sem_h0, sem_h1, sem_o0, sem_o1):
    wid = lax.axis_index("s") * NC + lax.axis_index("c")
    base = wid * ROWS_PER_W
    iota = lax.iota(jnp.int32, LANES)
    obufs = (obuf0, obuf1)
    hbufs = (hbuf0, hbuf1)
    sems_g = (sem_g0, sem_g1)
    sems_h = (sem_h0, sem_h1)
    sems_o = (sem_o0, sem_o1)

    cps = []
    for f, ref in enumerate((adid, adv, dep, pos, desc, uid, qid, kid, tid)):
        cp = pltpu.make_async_copy(
            ref.at[pl.ds(base, ROWS_PER_W)],
            featb.at[pl.ds(f * ROWS_PER_W, ROWS_PER_W)], sem_in)
        cp.start()
        cps.append(cp)
    for src, dst in ((ttok, ttkb), (qtok, qtkb)):
        cp = pltpu.make_async_copy(src.at[pl.ds(base, ROWS_PER_W)], dst, sem_in)
        cp.start()
        cps.append(cp)
    cp = pltpu.make_async_copy(adl.at[pl.ds(base, ROWS_PER_W)], adlb, sem_in)
    cp.start()
    cps.append(cp)
    for cp in cps:
        cp.wait()

    for s in range(NDIRECT, IDX_PITCH):
        idxs[s, :] = jnp.zeros((LANES,), jnp.int32)

    def build_group(g, carry):
        rb = g * GROUP
        for s, (fi, off) in enumerate(SCALAR_SLOTS):
            v = featb[pl.ds(fi * ROWS_PER_W + rb, GROUP)]
            idxs[s, :] = v % M + off * M
        for t in range(T_TOK):
            v = plsc.load_gather(ttkb, [rb + iota, jnp.full((LANES,), t, jnp.int32)])
            idxs[9 + t, :] = v % M + TTOK_OFF * M
        for t in range(T_TOK):
            v = plsc.load_gather(qtkb, [rb + iota, jnp.full((LANES,), t, jnp.int32)])
            idxs[29 + t, :] = v % M + QTOK_OFF * M
        for r in range(GROUP):
            for k in range(IDX_PITCH // LANES):
                v = plsc.load_gather(
                    idxs, [k * LANES + iota, jnp.full((LANES,), 1, jnp.int32) * r])
                idxd[pl.ds((rb + r) * IDX_PITCH + k * LANES, LANES)] = v
        return carry

    lax.fori_loop(0, ROWS_PER_W // GROUP, build_group, 0)

    def g_descs(c, p):
        return [pltpu.make_async_copy(
            table.at[idxd.at[pl.ds((c * CHUNK + r) * IDX_PITCH, NDIRECT)]],
            obufs[p].at[pl.ds(r * NSLOT, NDIRECT)], sems_g[p])
            for r in range(CHUNK)]

    def h_descs(c, p):
        return [pltpu.make_async_copy(
            table.at[adlb.at[c * CHUNK + r]],
            hbufs[p].at[pl.ds(r * L_HIST, L_HIST)], sems_h[p])
            for r in range(CHUNK)]

    def o_desc(c, p):
        return pltpu.make_async_copy(
            obufs[p], out.at[pl.ds((base + c * CHUNK) * NSLOT, ENT_PER_CHUNK)],
            sems_o[p])

    def pool(c, p):
        hb, ob = hbufs[p], obufs[p]

        def pool_q(q, carry):
            def pool_j(j, acc):
                return tuple(acc[k] + hb[q * L_HIST + j, pl.ds(k * LANES, LANES)]
                             for k in range(D // LANES))

            acc = lax.fori_loop(0, L_HIST, pool_j,
                                tuple(jnp.zeros((LANES,), jnp.float32)
                                      for _ in range(D // LANES)))
            for k in range(D // LANES):
                ob[q * NSLOT + NDIRECT, pl.ds(k * LANES, LANES)] = (
                    acc[k] * (1.0 / L_HIST))
            return carry

        lax.fori_loop(0, CHUNK, pool_q, 0)

    for p in (0, 1):
        for cp in g_descs(p, p) + h_descs(p, p):
            cp.start()

    def chunk_pair(k, carry):
        for p in (0, 1):
            c = 2 * k + p
            for cp in h_descs(c, p):
                cp.wait()
            pool(c, p)
            for cp in g_descs(c, p):
                cp.wait()
            o_desc(c, p).start()

            @pl.when(k < NCHUNK // 2 - 1)
            def _fire_h():
                for cp in h_descs(c + 2, p):
                    cp.start()

            o_desc(c, p).wait()

            @pl.when(k < NCHUNK // 2 - 1)
            def _fire_g():
                for cp in g_descs(c + 2, p):
                    cp.start()

        return carry

    lax.fori_loop(0, NCHUNK // 2, chunk_pair, 0)


FMT_CH = 16
FMT_PITCH = 208
XROWS = B * NSLOT // 2


def _fmt_body(x128, out, idxb, obuf, sem_g):
    wid = lax.axis_index("s") * NC + lax.axis_index("c")
    iota = lax.iota(jnp.int32, LANES)

    def build(c, carry):
        blk = wid * FMT_CH + c
        for k in range(FMT_PITCH // LANES):
            n = k * LANES + iota
            v = 200 * blk + 25 * (n & 7) + (n >> 3)
            v = jnp.minimum(v, XROWS - 1)
            idxb[pl.ds(c * FMT_PITCH + k * LANES, LANES)] = v
        return carry

    lax.fori_loop(0, FMT_CH, build, 0)

    def chunk(c, carry):
        blk = wid * FMT_CH + c
        cps = []
        for j in range(25):
            cp = pltpu.make_async_copy(
                x128.at[idxb.at[pl.ds(c * FMT_PITCH + 8 * j, 8)]],
                obuf.at[:, pl.ds(128 * j, 128)], sem_g)
            cp.start()
            cps.append(cp)
        for cp in cps:
            cp.wait()
        pltpu.sync_copy(obuf, out.at[pl.ds(8 * blk, 8)])
        return carry

    lax.fori_loop(0, FMT_CH, chunk, 0)


def kernel(AdID, AdvertiserID, Depth, Position, DescriptionID, user_id,
           QueryID, KeywordID, TitleID, TitleToken, QueryToken, AdIDList, emb_table):
    i32 = jnp.int32
    args = [a.astype(i32) for a in
            (AdID, AdvertiserID, Depth, Position, DescriptionID, user_id,
             QueryID, KeywordID, TitleID, TitleToken, QueryToken, AdIDList)]
    mesh = plsc.VectorSubcoreMesh(core_axis_name="c", subcore_axis_name="s",
                                  num_cores=NC, num_subcores=NS)
    out = pl.kernel(
        _gather_body,
        out_type=jax.ShapeDtypeStruct((B * NSLOT, D), jnp.float32),
        mesh=mesh,
        compiler_params=pltpu.CompilerParams(needs_layout_passes=False,
                                             use_tc_tiling_on_sc=False,
                                             skip_device_barrier=True),
        scratch_types=[
            pltpu.VMEM((9 * ROWS_PER_W,), i32),
            pltpu.VMEM((ROWS_PER_W, T_TOK), i32),
            pltpu.VMEM((ROWS_PER_W, T_TOK), i32),
            pltpu.VMEM((ROWS_PER_W, L_HIST), i32),
            pltpu.VMEM((IDX_PITCH, GROUP), i32),
            pltpu.VMEM((ROWS_PER_W * IDX_PITCH,), i32),
            pltpu.VMEM((ENT_PER_CHUNK, D), jnp.float32),
            pltpu.VMEM((ENT_PER_CHUNK, D), jnp.float32),
            pltpu.VMEM((ENT_PER_CHUNK, D), jnp.float32),
            pltpu.VMEM((ENT_PER_CHUNK, D), jnp.float32),
            pltpu.SemaphoreType.DMA,
            pltpu.SemaphoreType.DMA,
            pltpu.SemaphoreType.DMA,
            pltpu.SemaphoreType.DMA,
            pltpu.SemaphoreType.DMA,
            pltpu.SemaphoreType.DMA,
            pltpu.SemaphoreType.DMA,
        ],
    )(*args, emb_table)
    x128 = out.reshape(XROWS, 128)
    z = pl.kernel(
        _fmt_body,
        out_type=jax.ShapeDtypeStruct((B, NSLOT * D), jnp.float32),
        mesh=plsc.VectorSubcoreMesh(core_axis_name="c", subcore_axis_name="s",
                                    num_cores=NC, num_subcores=NS),
        compiler_params=pltpu.CompilerParams(needs_layout_passes=False,
                                             use_tc_tiling_on_sc=True,
                                             skip_device_barrier=True),
        scratch_types=[
            pltpu.VMEM((FMT_CH * FMT_PITCH,), i32),
            pltpu.VMEM((8, NSLOT * D), jnp.float32),
            pltpu.SemaphoreType.DMA,
        ],
    )(x128)
    return z


# speedup vs baseline: 2.0245x; 1.8513x over previous
import jax
import jax.numpy as jnp
from jax import lax
from jax.experimental import pallas as pl
from jax.experimental.pallas import tpu as pltpu
from jax.experimental.pallas import tpu_sc as plsc

B = 4096
L_HIST = 50
T_TOK = 20
V = 1_000_000
D = 64
M = 100_000

NC, NS, LANES = 2, 16, 16
NW = NC * NS
ROWS_PER_W = B // NW
CHUNK = 8
NCHUNK = ROWS_PER_W // CHUNK
NDIRECT = 49
NSLOT = 50
GROUP = 16
ENT_PER_CHUNK = CHUNK * NSLOT
IDX_PITCH = 112
HIST_OFF = 56

SCALAR_SLOTS = ((0, 0), (1, 1), (2, 0), (3, 0), (4, 2), (5, 3), (6, 4), (7, 5), (8, 6))
TTOK_OFF = 7
QTOK_OFF = 8


def _remap(v):
    p = v & (TW - 1)
    ph = p >> 13
    return (v - p) + ((p & (TH - 1)) << 1) + ph


def _gather_body(adid, adv, dep, pos, desc, uid, qid, kid, tid, ttok, qtok,
                 adl, table, out, featb, ttkb, qtkb, adlb, idxs, idxd,
                 obuf0, obuf1, hbuf,
                 sem_in, sem_g0, sem_g1, sem_h, sem_o0, sem_o1):
    wid = lax.axis_index("s") * NC + lax.axis_index("c")
    base = wid * ROWS_PER_W
    iota = lax.iota(jnp.int32, LANES)
    obufs = (obuf0, obuf1)
    sems_g = (sem_g0, sem_g1)
    sems_o = (sem_o0, sem_o1)

    cps = []
    for f, ref in enumerate((adid, adv, dep, pos, desc, uid, qid, kid, tid)):
        cp = pltpu.make_async_copy(
            ref.at[pl.ds(base, ROWS_PER_W)],
            featb.at[pl.ds(f * ROWS_PER_W, ROWS_PER_W)], sem_in)
        cp.start()
        cps.append(cp)
    for src, dst in ((ttok, ttkb), (qtok, qtkb)):
        cp = pltpu.make_async_copy(src.at[pl.ds(base, ROWS_PER_W)], dst, sem_in)
        cp.start()
        cps.append(cp)
    cp = pltpu.make_async_copy(adl.at[pl.ds(base, ROWS_PER_W)], adlb, sem_in)
    cp.start()
    cps.append(cp)
    for cp in cps:
        cp.wait()

    for s in range(NDIRECT, 4 * LANES):
        idxs[s, :] = jnp.zeros((LANES,), jnp.int32)

    def build_group(g, carry):
        rb = g * GROUP
        for s, (fi, off) in enumerate(SCALAR_SLOTS):
            v = featb[pl.ds(fi * ROWS_PER_W + rb, GROUP)]
            idxs[s, :] = _remap(v % M + off * M)
        for t in range(T_TOK):
            v = plsc.load_gather(ttkb, [rb + iota, jnp.full((LANES,), t, jnp.int32)])
            idxs[9 + t, :] = _remap(v % M + TTOK_OFF * M)
        for t in range(T_TOK):
            v = plsc.load_gather(qtkb, [rb + iota, jnp.full((LANES,), t, jnp.int32)])
            idxs[29 + t, :] = _remap(v % M + QTOK_OFF * M)
        def row_build(r, carry2):
            ones = jnp.full((LANES,), 1, jnp.int32)
            for k in range(4):
                v = plsc.load_gather(idxs, [k * LANES + iota, ones * r])
                idxd[pl.ds((rb + r) * IDX_PITCH + k * LANES, LANES)] = v
            for k in range(4):
                cols = jnp.minimum(k * LANES + iota, L_HIST - 1)
                v = plsc.load_gather(adlb, [ones * (rb + r), cols])
                idxd[pl.ds((rb + r) * IDX_PITCH + HIST_OFF + k * LANES, LANES)] = (
                    _remap(v))
            return carry2

        lax.fori_loop(0, GROUP, row_build, 0)
        return carry

    lax.fori_loop(0, ROWS_PER_W // GROUP, build_group, 0)

    def g_desc(c, r, p):
        return pltpu.make_async_copy(
            table.at[idxd.at[pl.ds((c * CHUNK + r) * IDX_PITCH, NDIRECT)]],
            obufs[p].at[pl.ds(r * NSLOT, NDIRECT)], sems_g[p])

    def h_desc(c, r):
        return pltpu.make_async_copy(
            table.at[idxd.at[pl.ds((c * CHUNK + r) * IDX_PITCH + HIST_OFF,
                                   L_HIST)]],
            hbuf.at[pl.ds(r * L_HIST, L_HIST)], sem_h)

    def rows_fori(fn):
        lax.fori_loop(0, CHUNK, lambda r, carry: (fn(r), carry)[1], 0)

    def o_desc(c, p):
        return pltpu.make_async_copy(
            obufs[p], out.at[pl.ds((base + c * CHUNK) * NSLOT, ENT_PER_CHUNK)],
            sems_o[p])

    def pool(c, p):
        ob = obufs[p]

        def pool_q(q, carry):
            def pool_j(j, acc):
                return tuple(acc[k] + hbuf[q * L_HIST + j, pl.ds(k * LANES, LANES)]
                             for k in range(D // LANES))

            acc = lax.fori_loop(0, L_HIST, pool_j,
                                tuple(jnp.zeros((LANES,), jnp.float32)
                                      for _ in range(D // LANES)))
            for k in range(D // LANES):
                ob[q * NSLOT + NDIRECT, pl.ds(k * LANES, LANES)] = (
                    acc[k] * (1.0 / L_HIST))
            return carry

        lax.fori_loop(0, CHUNK, pool_q, 0)

    for p in (0, 1):
        rows_fori(lambda r, p=p: g_desc(p, r, p).start())
    rows_fori(lambda r: h_desc(0, r).start())

    def chunk_pair(k, carry):
        for p in (0, 1):
            c = 2 * k + p
            rows_fori(lambda r: h_desc(c, r).wait())
            pool(c, p)
            if p == 0:
                rows_fori(lambda r: h_desc(c + 1, r).start())
            else:
                @pl.when(k < NCHUNK // 2 - 1)
                def _fire_h():
                    rows_fori(lambda r: h_desc(c + 1, r).start())
            rows_fori(lambda r: g_desc(c, r, p).wait())
            ocp = o_desc(c, p)
            ocp.start()
            ocp.wait()

            @pl.when(k < NCHUNK // 2 - 1)
            def _fire_g():
                rows_fori(lambda r: g_desc(c + 2, r, p).start())

        return carry

    lax.fori_loop(0, NCHUNK // 2, chunk_pair, 0)


TW = 16384
TH = TW // 2
NBLK = -(-V // TW)
VPAD = NBLK * TW


def _detile_body(tt_ref, out_ref):
    x = tt_ref[...]
    lo = jnp.transpose(x[:, :TH], (1, 0))
    hi = jnp.transpose(x[:, TH:], (1, 0))
    out_ref[...] = jnp.concatenate([lo, hi], axis=1)


FMT_CH = 16
FMT_PITCH = 208
XROWS = B * NSLOT // 2


def _fmt_body(x128, out, idxb, obuf, sem_g):
    wid = lax.axis_index("s") * NC + lax.axis_index("c")
    iota = lax.iota(jnp.int32, LANES)

    def build(c, carry):
        blk = wid * FMT_CH + c
        for k in range(FMT_PITCH // LANES):
            n = k * LANES + iota
            v = 200 * blk + 25 * (n & 7) + (n >> 3)
            v = jnp.minimum(v, XROWS - 1)
            idxb[pl.ds(c * FMT_PITCH + k * LANES, LANES)] = v
        return carry

    lax.fori_loop(0, FMT_CH, build, 0)

    def chunk(c, carry):
        blk = wid * FMT_CH + c
        cps = []
        for j in range(25):
            cp = pltpu.make_async_copy(
                x128.at[idxb.at[pl.ds(c * FMT_PITCH + 8 * j, 8)]],
                obuf.at[:, pl.ds(128 * j, 128)], sem_g)
            cp.start()
            cps.append(cp)
        for cp in cps:
            cp.wait()
        pltpu.sync_copy(obuf, out.at[pl.ds(8 * blk, 8)])
        return carry

    lax.fori_loop(0, FMT_CH, chunk, 0)


def kernel(AdID, AdvertiserID, Depth, Position, DescriptionID, user_id,
           QueryID, KeywordID, TitleID, TitleToken, QueryToken, AdIDList, emb_table):
    i32 = jnp.int32
    args = [a.astype(i32) for a in
            (AdID, AdvertiserID, Depth, Position, DescriptionID, user_id,
             QueryID, KeywordID, TitleID, TitleToken, QueryToken, AdIDList)]
    tableL = pl.pallas_call(
        _detile_body,
        grid=(NBLK,),
        in_specs=[pl.BlockSpec((D, TW), lambda j: (0, j))],
        out_specs=pl.BlockSpec((TH, 2 * D), lambda j: (j, 0)),
        out_shape=jax.ShapeDtypeStruct((VPAD // 2, 2 * D), jnp.float32),
    )(emb_table.T)
    table_lin = tableL.reshape(VPAD, D)
    mesh = plsc.VectorSubcoreMesh(core_axis_name="c", subcore_axis_name="s",
                                  num_cores=NC, num_subcores=NS)
    out = pl.kernel(
        _gather_body,
        out_type=jax.ShapeDtypeStruct((B * NSLOT, D), jnp.float32),
        mesh=mesh,
        compiler_params=pltpu.CompilerParams(needs_layout_passes=False,
                                             use_tc_tiling_on_sc=False,
                                             skip_device_barrier=True),
        scratch_types=[
            pltpu.VMEM((9 * ROWS_PER_W,), i32),
            pltpu.VMEM((ROWS_PER_W, T_TOK), i32),
            pltpu.VMEM((ROWS_PER_W, T_TOK), i32),
            pltpu.VMEM((ROWS_PER_W, L_HIST), i32),
            pltpu.VMEM((4 * LANES, GROUP), i32),
            pltpu.VMEM((ROWS_PER_W * IDX_PITCH + LANES,), i32),
            pltpu.VMEM((ENT_PER_CHUNK, D), jnp.float32),
            pltpu.VMEM((ENT_PER_CHUNK, D), jnp.float32),
            pltpu.VMEM((ENT_PER_CHUNK, D), jnp.float32),
            pltpu.SemaphoreType.DMA,
            pltpu.SemaphoreType.DMA,
            pltpu.SemaphoreType.DMA,
            pltpu.SemaphoreType.DMA,
            pltpu.SemaphoreType.DMA,
            pltpu.SemaphoreType.DMA,
        ],
    )(*args, table_lin)
    x128 = out.reshape(XROWS, 128)
    z = pl.kernel(
        _fmt_body,
        out_type=jax.ShapeDtypeStruct((B, NSLOT * D), jnp.float32),
        mesh=plsc.VectorSubcoreMesh(core_axis_name="c", subcore_axis_name="s",
                                    num_cores=NC, num_subcores=NS),
        compiler_params=pltpu.CompilerParams(needs_layout_passes=False,
                                             use_tc_tiling_on_sc=True,
                                             skip_device_barrier=True),
        scratch_types=[
            pltpu.VMEM((FMT_CH * FMT_PITCH,), i32),
            pltpu.VMEM((8, NSLOT * D), jnp.float32),
            pltpu.SemaphoreType.DMA,
        ],
    )(x128)
    return z


# speedup vs baseline: 2.0801x; 1.0275x over previous
import jax
import jax.numpy as jnp
from jax import lax
from jax.experimental import pallas as pl
from jax.experimental.pallas import tpu as pltpu
from jax.experimental.pallas import tpu_sc as plsc

B = 4096
L_HIST = 50
T_TOK = 20
V = 1_000_000
D = 64
M = 100_000

NC, NS, LANES = 2, 16, 16
NW = NC * NS
ROWS_PER_W = B // NW
CHUNK = 8
NCHUNK = ROWS_PER_W // CHUNK
NDIRECT = 49
NSLOT = 50
GROUP = 16
ENT_PER_CHUNK = CHUNK * NSLOT
IDX_PITCH = 112
HIST_OFF = 56

SCALAR_SLOTS = ((0, 0), (1, 1), (2, 0), (3, 0), (4, 2), (5, 3), (6, 4), (7, 5), (8, 6))
TTOK_OFF = 7
QTOK_OFF = 8


def _remap(v):
    p = v & (TW - 1)
    ph = p >> 13
    return (v - p) + ((p & (TH - 1)) << 1) + ph


def _gather_body(adid, adv, dep, pos, desc, uid, qid, kid, tid, ttok, qtok,
                 adl, table, out, featb, ttkb, qtkb, adlb, idxs, idxd,
                 obuf0, obuf1, hbuf,
                 sem_in, sem_g0, sem_g1, sem_h, sem_o0, sem_o1):
    wid = lax.axis_index("s") * NC + lax.axis_index("c")
    base = wid * ROWS_PER_W
    iota = lax.iota(jnp.int32, LANES)
    obufs = (obuf0, obuf1)
    sems_g = (sem_g0, sem_g1)
    sems_o = (sem_o0, sem_o1)

    cps = []
    for f, ref in enumerate((adid, adv, dep, pos, desc, uid, qid, kid, tid)):
        cp = pltpu.make_async_copy(
            ref.at[pl.ds(base, ROWS_PER_W)],
            featb.at[pl.ds(f * ROWS_PER_W, ROWS_PER_W)], sem_in)
        cp.start()
        cps.append(cp)
    for src, dst in ((ttok, ttkb), (qtok, qtkb)):
        cp = pltpu.make_async_copy(src.at[pl.ds(base, ROWS_PER_W)], dst, sem_in)
        cp.start()
        cps.append(cp)
    cp = pltpu.make_async_copy(adl.at[pl.ds(base, ROWS_PER_W)], adlb, sem_in)
    cp.start()
    cps.append(cp)
    for cp in cps:
        cp.wait()

    for s in range(NDIRECT, 4 * LANES):
        idxs[s, :] = jnp.zeros((LANES,), jnp.int32)

    def build_group(g, carry):
        rb = g * GROUP
        for s, (fi, off) in enumerate(SCALAR_SLOTS):
            v = featb[pl.ds(fi * ROWS_PER_W + rb, GROUP)]
            idxs[s, :] = _remap(v % M + off * M)
        for t in range(T_TOK):
            v = plsc.load_gather(ttkb, [rb + iota, jnp.full((LANES,), t, jnp.int32)])
            idxs[9 + t, :] = _remap(v % M + TTOK_OFF * M)
        for t in range(T_TOK):
            v = plsc.load_gather(qtkb, [rb + iota, jnp.full((LANES,), t, jnp.int32)])
            idxs[29 + t, :] = _remap(v % M + QTOK_OFF * M)
        def row_build(r, carry2):
            ones = jnp.full((LANES,), 1, jnp.int32)
            for k in range(4):
                v = plsc.load_gather(idxs, [k * LANES + iota, ones * r])
                idxd[pl.ds((rb + r) * IDX_PITCH + k * LANES, LANES)] = v
            for k in range(4):
                cols = jnp.minimum(k * LANES + iota, L_HIST - 1)
                v = plsc.load_gather(adlb, [ones * (rb + r), cols])
                idxd[pl.ds((rb + r) * IDX_PITCH + HIST_OFF + k * LANES, LANES)] = (
                    _remap(v))
            return carry2

        lax.fori_loop(0, GROUP, row_build, 0)
        return carry

    lax.fori_loop(0, ROWS_PER_W // GROUP, build_group, 0)

    def g_desc(c, r, p):
        return pltpu.make_async_copy(
            table.at[idxd.at[pl.ds((c * CHUNK + r) * IDX_PITCH, NDIRECT)]],
            obufs[p].at[pl.ds(r * NSLOT, NDIRECT)], sems_g[p])

    def h_desc(c, r):
        return pltpu.make_async_copy(
            table.at[idxd.at[pl.ds((c * CHUNK + r) * IDX_PITCH + HIST_OFF,
                                   L_HIST)]],
            hbuf.at[pl.ds(r * L_HIST, L_HIST)], sem_h)

    def rows_fori(fn):
        lax.fori_loop(0, CHUNK, lambda r, carry: (fn(r), carry)[1], 0)

    def o_desc(c, p):
        return pltpu.make_async_copy(
            obufs[p], out.at[pl.ds((base + c * CHUNK) * NSLOT, ENT_PER_CHUNK)],
            sems_o[p])

    def pool(c, p):
        ob = obufs[p]

        def pool_q(q, carry):
            def pool_j(j, acc):
                return tuple(acc[k] + hbuf[q * L_HIST + j, pl.ds(k * LANES, LANES)]
                             for k in range(D // LANES))

            acc = lax.fori_loop(0, L_HIST, pool_j,
                                tuple(jnp.zeros((LANES,), jnp.float32)
                                      for _ in range(D // LANES)))
            for k in range(D // LANES):
                ob[q * NSLOT + NDIRECT, pl.ds(k * LANES, LANES)] = (
                    acc[k] * (1.0 / L_HIST))
            return carry

        lax.fori_loop(0, CHUNK, pool_q, 0)

    for p in (0, 1):
        rows_fori(lambda r, p=p: g_desc(p, r, p).start())
    rows_fori(lambda r: h_desc(0, r).start())

    def chunk_pair(k, carry):
        for p in (0, 1):
            c = 2 * k + p
            rows_fori(lambda r: h_desc(c, r).wait())
            pool(c, p)
            if p == 0:
                rows_fori(lambda r: h_desc(c + 1, r).start())
            else:
                @pl.when(k < NCHUNK // 2 - 1)
                def _fire_h():
                    rows_fori(lambda r: h_desc(c + 1, r).start())
            rows_fori(lambda r: g_desc(c, r, p).wait())
            ocp = o_desc(c, p)
            ocp.start()
            ocp.wait()

            @pl.when(k < NCHUNK // 2 - 1)
            def _fire_g():
                rows_fori(lambda r: g_desc(c + 2, r, p).start())

        return carry

    lax.fori_loop(0, NCHUNK // 2, chunk_pair, 0)


TW = 16384
TH = TW // 2
NBLK = -(-V // TW)
VPAD = NBLK * TW


def _detile_body(tt_ref, out_ref):
    x = tt_ref[...]
    lo = jnp.transpose(x[:, :TH], (1, 0))
    hi = jnp.transpose(x[:, TH:], (1, 0))
    out_ref[...] = jnp.concatenate([lo, hi], axis=1)


FMT_CH = 16
FMT_PITCH = 208
XROWS = B * NSLOT // 2


def _fmt_body(x128, out, idxb, obuf0, obuf1, sem_g0, sem_g1, sem_o0, sem_o1):
    wid = lax.axis_index("s") * NC + lax.axis_index("c")
    iota = lax.iota(jnp.int32, LANES)

    def build(c, carry):
        blk = wid * FMT_CH + c
        for k in range(FMT_PITCH // LANES):
            n = k * LANES + iota
            v = 200 * blk + 25 * (n & 7) + (n >> 3)
            v = jnp.minimum(v, XROWS - 1)
            idxb[pl.ds(c * FMT_PITCH + k * LANES, LANES)] = v
        return carry

    lax.fori_loop(0, FMT_CH, build, 0)

    obufs = (obuf0, obuf1)
    sems_g = (sem_g0, sem_g1)
    sems_o = (sem_o0, sem_o1)

    def fire(c, p):
        def f(j, carry):
            pltpu.make_async_copy(
                x128.at[idxb.at[pl.ds(c * FMT_PITCH + 8 * j, 8)]],
                obufs[p].at[:, pl.ds(128 * j, 128)], sems_g[p]).start()
            return carry

        lax.fori_loop(0, 25, f, 0)

    def drain(c, p):
        def f(j, carry):
            pltpu.make_async_copy(
                x128.at[idxb.at[pl.ds(c * FMT_PITCH + 8 * j, 8)]],
                obufs[p].at[:, pl.ds(128 * j, 128)], sems_g[p]).wait()
            return carry

        lax.fori_loop(0, 25, f, 0)

    def o_desc(c, p):
        blk = wid * FMT_CH + c
        return pltpu.make_async_copy(obufs[p], out.at[pl.ds(8 * blk, 8)],
                                     sems_o[p])

    fire(0, 0)
    fire(1, 1)

    def chunk_pair(k, carry):
        for p in (0, 1):
            c = 2 * k + p
            drain(c, p)
            ocp = o_desc(c, p)
            ocp.start()
            ocp.wait()

            @pl.when(k < FMT_CH // 2 - 1)
            def _next():
                fire(c + 2, p)

        return carry

    lax.fori_loop(0, FMT_CH // 2, chunk_pair, 0)


def kernel(AdID, AdvertiserID, Depth, Position, DescriptionID, user_id,
           QueryID, KeywordID, TitleID, TitleToken, QueryToken, AdIDList, emb_table):
    i32 = jnp.int32
    args = [a.astype(i32) for a in
            (AdID, AdvertiserID, Depth, Position, DescriptionID, user_id,
             QueryID, KeywordID, TitleID, TitleToken, QueryToken, AdIDList)]
    tableL = pl.pallas_call(
        _detile_body,
        grid=(NBLK,),
        in_specs=[pl.BlockSpec((D, TW), lambda j: (0, j))],
        out_specs=pl.BlockSpec((TH, 2 * D), lambda j: (j, 0)),
        out_shape=jax.ShapeDtypeStruct((VPAD // 2, 2 * D), jnp.float32),
    )(emb_table.T)
    table_lin = tableL.reshape(VPAD, D)
    mesh = plsc.VectorSubcoreMesh(core_axis_name="c", subcore_axis_name="s",
                                  num_cores=NC, num_subcores=NS)
    out = pl.kernel(
        _gather_body,
        out_type=jax.ShapeDtypeStruct((B * NSLOT, D), jnp.float32),
        mesh=mesh,
        compiler_params=pltpu.CompilerParams(needs_layout_passes=False,
                                             use_tc_tiling_on_sc=False,
                                             skip_device_barrier=True),
        scratch_types=[
            pltpu.VMEM((9 * ROWS_PER_W,), i32),
            pltpu.VMEM((ROWS_PER_W, T_TOK), i32),
            pltpu.VMEM((ROWS_PER_W, T_TOK), i32),
            pltpu.VMEM((ROWS_PER_W, L_HIST), i32),
            pltpu.VMEM((4 * LANES, GROUP), i32),
            pltpu.VMEM((ROWS_PER_W * IDX_PITCH + LANES,), i32),
            pltpu.VMEM((ENT_PER_CHUNK, D), jnp.float32),
            pltpu.VMEM((ENT_PER_CHUNK, D), jnp.float32),
            pltpu.VMEM((ENT_PER_CHUNK, D), jnp.float32),
            pltpu.SemaphoreType.DMA,
            pltpu.SemaphoreType.DMA,
            pltpu.SemaphoreType.DMA,
            pltpu.SemaphoreType.DMA,
            pltpu.SemaphoreType.DMA,
            pltpu.SemaphoreType.DMA,
        ],
    )(*args, table_lin)
    x128 = out.reshape(XROWS, 128)
    z = pl.kernel(
        _fmt_body,
        out_type=jax.ShapeDtypeStruct((B, NSLOT * D), jnp.float32),
        mesh=plsc.VectorSubcoreMesh(core_axis_name="c", subcore_axis_name="s",
                                    num_cores=NC, num_subcores=NS),
        compiler_params=pltpu.CompilerParams(needs_layout_passes=False,
                                             use_tc_tiling_on_sc=True,
                                             skip_device_barrier=True),
        scratch_types=[
            pltpu.VMEM((FMT_CH * FMT_PITCH,), i32),
            pltpu.VMEM((8, NSLOT * D), jnp.float32),
            pltpu.VMEM((8, NSLOT * D), jnp.float32),
            pltpu.SemaphoreType.DMA,
            pltpu.SemaphoreType.DMA,
            pltpu.SemaphoreType.DMA,
            pltpu.SemaphoreType.DMA,
        ],
    )(x128)
    return z
